# Initial kernel scaffold; baseline (speedup 1.0000x reference)
import jax
import jax.numpy as jnp
from jax.experimental import pallas as pl


def kernel(x, embedding_weight):
    raise NotImplementedError("write your pallas kernel here")


# speedup vs baseline: 1.5603x; 1.5603x over previous
import functools

import jax
import jax.numpy as jnp
from jax import lax
from jax.experimental import pallas as pl
from jax.experimental.pallas import tpu as pltpu
from jax.experimental.pallas import tpu_sc as plsc

BATCH = 16384
N_FIELDS = 26
EMBED_DIM = 32

NUM_CORES = 2
NUM_SUBCORES = 16
NW = NUM_CORES * NUM_SUBCORES

B_TOTAL = BATCH * N_FIELDS
N_PER_W = B_TOTAL // NW
G = 128
ROWS_PER_W = N_PER_W // G
CHUNK_ROWS = 8
CHUNK = CHUNK_ROWS * G
N_OUTER = ROWS_PER_W // CHUNK_ROWS

_mesh = plsc.VectorSubcoreMesh(core_axis_name="c", subcore_axis_name="s")


@functools.partial(
    pl.kernel,
    out_type=jax.ShapeDtypeStruct((B_TOTAL, EMBED_DIM), jnp.float32),
    mesh=_mesh,
    scratch_types=[
        pltpu.VMEM((ROWS_PER_W, G), jnp.int32),
        pltpu.VMEM((CHUNK, EMBED_DIM), jnp.float32),
        pltpu.SemaphoreType.DMA,
    ],
    compiler_params=pltpu.CompilerParams(use_tc_tiling_on_sc=False),
)
def _emb_lookup(idx_hbm, table_hbm, out_hbm, idx_v, rows_v, sem):
    wid = lax.axis_index("s") * NUM_CORES + lax.axis_index("c")
    row_base = wid * ROWS_PER_W
    out_base = wid * N_PER_W

    pltpu.sync_copy(idx_hbm.at[pl.ds(row_base, ROWS_PER_W)], idx_v)

    def outer(i, _):
        copies = [
            pltpu.async_copy(
                table_hbm.at[idx_v.at[i * CHUNK_ROWS + j]],
                rows_v.at[pl.ds(j * G, G)],
                sem,
            )
            for j in range(CHUNK_ROWS)
        ]
        for c in copies:
            c.wait()
        pltpu.sync_copy(rows_v, out_hbm.at[pl.ds(out_base + i * CHUNK, CHUNK)])
        return 0

    lax.fori_loop(0, N_OUTER, outer, 0)


def kernel(x, embedding_weight):
    idx = x.reshape(B_TOTAL // G, G).astype(jnp.int32)
    out = _emb_lookup(idx, embedding_weight)
    return out.reshape(BATCH, N_FIELDS, EMBED_DIM)
